# Initial kernel scaffold; baseline (speedup 1.0000x reference)
import functools

import jax
import jax.numpy as jnp
from jax import lax
from jax.experimental import pallas as pl
from jax.experimental.pallas import tpu as pltpu
from jax.experimental.pallas import tpu_sc as plsc

N = 10000
E = 320000
D = 128
T = 16
NB = 8
CUT = 4.0

NC = 2
NS = 16
NW = NC * NS
EPW = E // NW
CH = 80
NCH = EPW // CH

F32 = jnp.float32
I32 = jnp.int32

_MESH = plsc.VectorSubcoreMesh(
    core_axis_name="c", subcore_axis_name="s", num_cores=NC, num_subcores=NS)


def _worker_id():
    return lax.axis_index("s") * NC + lax.axis_index("c")


def _ew_mul(dst, a, b):
    def row(r, _):
        for rr in range(2):
            ri = r * 2 + rr
            for k in range(D // 16):
                sl = pl.ds(k * 16, 16)
                dst[ri, sl] = a[ri, sl] * b[ri, sl]
        return 0
    lax.fori_loop(0, CH // 2, row, 0)


@functools.partial(
    pl.kernel,
    out_type=(jax.ShapeDtypeStruct((NW, NCH, CH, 16), F32),
              jax.ShapeDtypeStruct((NW, NCH, CH, 16), F32)),
    mesh=_MESH,
    scratch_types=[
        pltpu.VMEM((NCH, CH), I32),
        pltpu.VMEM((NCH, CH), I32),
        pltpu.VMEM((CH, 16), F32),
        pltpu.VMEM((CH, 16), F32),
        pltpu.SemaphoreType.DMA,
        pltpu.SemaphoreType.DMA,
    ],
)
def _sc_gather_pos(pos_hbm, j_hbm, i_hbm, pj_hbm, pi_hbm, jv, iv, bj, bi, s1, s2):
    w = _worker_id()
    pltpu.sync_copy(j_hbm.at[w], jv)
    pltpu.sync_copy(i_hbm.at[w], iv)

    def body(c, _):
        cj = pltpu.async_copy(pos_hbm.at[jv.at[c]], bj, s1)
        ci = pltpu.async_copy(pos_hbm.at[iv.at[c]], bi, s2)
        cj.wait()
        ci.wait()
        pltpu.sync_copy(bj, pj_hbm.at[w, c])
        pltpu.sync_copy(bi, pi_hbm.at[w, c])
        return 0
    lax.fori_loop(0, NCH, body, 0)


@functools.partial(
    pl.kernel,
    out_type=jax.ShapeDtypeStruct((NC, N, D), F32),
    mesh=_MESH,
    scratch_types=[
        pltpu.VMEM((NCH, CH), I32),
        pltpu.VMEM((NCH, CH), I32),
        pltpu.VMEM((CH, D), F32),
        pltpu.VMEM((CH, D), F32),
        pltpu.VMEM((CH, D), F32),
        pltpu.VMEM_SHARED((N, D), F32),
        pltpu.SemaphoreType.DMA,
        pltpu.SemaphoreType.DMA,
    ],
)
def _sc_fwd_msg(w_hbm, hm_hbm, j_hbm, i_hbm, z_hbm, out_hbm,
                jv, iv, wbuf, hmbuf, msgbuf, acc, s1, s2):
    cid = lax.axis_index("c")
    sid = lax.axis_index("s")
    w = sid * NC + cid

    @pl.when(sid == 0)
    def _():
        pltpu.sync_copy(z_hbm, acc)
    pltpu.sync_copy(j_hbm.at[w], jv)
    pltpu.sync_copy(i_hbm.at[w], iv)
    plsc.subcore_barrier()

    def body(c, _):
        cw = pltpu.async_copy(w_hbm.at[w, c], wbuf, s1)
        chm = pltpu.async_copy(hm_hbm.at[jv.at[c]], hmbuf, s2)
        cw.wait()
        chm.wait()
        _ew_mul(msgbuf, wbuf, hmbuf)
        pltpu.sync_copy(msgbuf, acc.at[iv.at[c]], add=True)
        return 0
    lax.fori_loop(0, NCH, body, 0)

    plsc.subcore_barrier()

    @pl.when(sid == 0)
    def _():
        pltpu.sync_copy(acc, out_hbm.at[cid])


@functools.partial(
    pl.kernel,
    out_type=(jax.ShapeDtypeStruct((NW, NCH, CH, D), F32),
              jax.ShapeDtypeStruct((NC, N, D), F32)),
    mesh=_MESH,
    scratch_types=[
        pltpu.VMEM((NCH, CH), I32),
        pltpu.VMEM((NCH, CH), I32),
        pltpu.VMEM((CH, D), F32),
        pltpu.VMEM((CH, D), F32),
        pltpu.VMEM((CH, D), F32),
        pltpu.VMEM((CH, D), F32),
        pltpu.VMEM_SHARED((N, D), F32),
        pltpu.SemaphoreType.DMA,
        pltpu.SemaphoreType.DMA,
        pltpu.SemaphoreType.DMA,
    ],
)
def _sc_bwd2(w_hbm, hm_hbm, dg_hbm, j_hbm, i_hbm, z_hbm, dw_hbm, out_hbm,
             jv, iv, wbuf, hmbuf, dgbuf, dwbuf, acc, s1, s2, s3):
    cid = lax.axis_index("c")
    sid = lax.axis_index("s")
    w = sid * NC + cid

    @pl.when(sid == 0)
    def _():
        pltpu.sync_copy(z_hbm, acc)
    pltpu.sync_copy(j_hbm.at[w], jv)
    pltpu.sync_copy(i_hbm.at[w], iv)
    plsc.subcore_barrier()

    def body(c, _):
        cw = pltpu.async_copy(w_hbm.at[w, c], wbuf, s1)
        chm = pltpu.async_copy(hm_hbm.at[jv.at[c]], hmbuf, s2)
        cdg = pltpu.async_copy(dg_hbm.at[iv.at[c]], dgbuf, s3)
        cw.wait()
        chm.wait()
        cdg.wait()
        _ew_mul(dwbuf, dgbuf, hmbuf)
        pltpu.sync_copy(dwbuf, dw_hbm.at[w, c])
        _ew_mul(hmbuf, dgbuf, wbuf)
        pltpu.sync_copy(hmbuf, acc.at[jv.at[c]], add=True)
        return 0
    lax.fori_loop(0, NCH, body, 0)

    plsc.subcore_barrier()

    @pl.when(sid == 0)
    def _():
        pltpu.sync_copy(acc, out_hbm.at[cid])


@functools.partial(
    pl.kernel,
    out_type=jax.ShapeDtypeStruct((NW, NCH, CH, D), F32),
    mesh=_MESH,
    scratch_types=[
        pltpu.VMEM((NCH, CH), I32),
        pltpu.VMEM((NCH, CH), I32),
        pltpu.VMEM((CH, D), F32),
        pltpu.VMEM((CH, D), F32),
        pltpu.VMEM((CH, D), F32),
        pltpu.SemaphoreType.DMA,
        pltpu.SemaphoreType.DMA,
    ],
)
def _sc_bwd1(hm_hbm, dg_hbm, j_hbm, i_hbm, dw_hbm,
             jv, iv, hmbuf, dgbuf, dwbuf, s1, s2):
    w = _worker_id()
    pltpu.sync_copy(j_hbm.at[w], jv)
    pltpu.sync_copy(i_hbm.at[w], iv)

    def body(c, _):
        chm = pltpu.async_copy(hm_hbm.at[jv.at[c]], hmbuf, s1)
        cdg = pltpu.async_copy(dg_hbm.at[iv.at[c]], dgbuf, s2)
        chm.wait()
        cdg.wait()
        _ew_mul(dwbuf, dgbuf, hmbuf)
        pltpu.sync_copy(dwbuf, dw_hbm.at[w, c])
        return 0
    lax.fori_loop(0, NCH, body, 0)


@functools.partial(
    pl.kernel,
    out_type=jax.ShapeDtypeStruct((NC, N, 16), F32),
    mesh=_MESH,
    scratch_types=[
        pltpu.VMEM((NCH, CH), I32),
        pltpu.VMEM((NCH, CH), I32),
        pltpu.VMEM((CH, 16), F32),
        pltpu.VMEM((CH, 16), F32),
        pltpu.VMEM_SHARED((N, 16), F32),
        pltpu.SemaphoreType.DMA,
        pltpu.SemaphoreType.DMA,
    ],
)
def _sc_force(ndv_hbm, dv_hbm, j_hbm, i_hbm, z_hbm, out_hbm,
              jv, iv, nbuf, dbuf, acc, s1, s2):
    cid = lax.axis_index("c")
    sid = lax.axis_index("s")
    w = sid * NC + cid

    @pl.when(sid == 0)
    def _():
        pltpu.sync_copy(z_hbm, acc)
    pltpu.sync_copy(j_hbm.at[w], jv)
    pltpu.sync_copy(i_hbm.at[w], iv)
    plsc.subcore_barrier()

    def body(c, _):
        cn = pltpu.async_copy(ndv_hbm.at[w, c], nbuf, s1)
        cd = pltpu.async_copy(dv_hbm.at[w, c], dbuf, s2)
        cn.wait()
        cd.wait()
        pltpu.sync_copy(nbuf, acc.at[jv.at[c]], add=True)
        pltpu.sync_copy(dbuf, acc.at[iv.at[c]], add=True)
        return 0
    lax.fori_loop(0, NCH, body, 0)

    plsc.subcore_barrier()

    @pl.when(sid == 0)
    def _():
        pltpu.sync_copy(acc, out_hbm.at[cid])


BLK = 3200


def _silu(x):
    return x * jax.nn.sigmoid(x)


def _radial_parts(pjv, piv):
    vec = pjv - piv
    r2 = jnp.sum(vec * vec, axis=1, keepdims=True) + 1e-6
    r = jnp.sqrt(r2)
    rinv = 1.0 / r
    rm = jnp.minimum(r, CUT)
    mask = (r < CUT).astype(F32)
    env = 0.5 * (jnp.cos(jnp.pi * rm / CUT) + 1.0) * mask
    nvec = lax.broadcasted_iota(F32, (1, NB), 1) + 1.0
    theta = (jnp.pi / CUT) * r * nvec
    return vec, r, rinv, rm, mask, env, nvec, theta


def _tc_radial_fwd(pj_ref, pi_ref, w1a_ref, w1b_ref, w2a_ref, w2b_ref,
                   vec_ref, w1_ref, w2_ref):
    vec, r, rinv, rm, mask, env, nvec, theta = _radial_parts(pj_ref[...], pi_ref[...])
    k = jnp.sqrt(2.0 / CUT)
    rb = k * jnp.sin(theta) * rinv * env
    a1 = jnp.dot(rb, w1a_ref[...], preferred_element_type=F32)
    w1_ref[...] = jnp.dot(_silu(a1), w1b_ref[...], preferred_element_type=F32)
    a2 = jnp.dot(rb, w2a_ref[...], preferred_element_type=F32)
    w2_ref[...] = jnp.dot(_silu(a2), w2b_ref[...], preferred_element_type=F32)
    vec_ref[...] = vec


def _tc_radial_bwd(vec_ref, dw1_ref, dw2_ref, w1a_ref, w1bT_ref, w1aT_ref,
                   w2a_ref, w2bT_ref, w2aT_ref, dv_ref, ndv_ref):
    z = jnp.zeros_like(vec_ref[...])
    vec, r, rinv, rm, mask, env, nvec, theta = _radial_parts(vec_ref[...], z)
    k = jnp.sqrt(2.0 / CUT)
    sth = jnp.sin(theta)
    cth = jnp.cos(theta)

    def dsilu(a):
        s = jax.nn.sigmoid(a)
        return s * (1.0 + a * (1.0 - s))

    rb = k * sth * rinv * env
    a1 = jnp.dot(rb, w1a_ref[...], preferred_element_type=F32)
    da1 = jnp.dot(dw1_ref[...], w1bT_ref[...], preferred_element_type=F32) * dsilu(a1)
    drb = jnp.dot(da1, w1aT_ref[...], preferred_element_type=F32)
    a2 = jnp.dot(rb, w2a_ref[...], preferred_element_type=F32)
    da2 = jnp.dot(dw2_ref[...], w2bT_ref[...], preferred_element_type=F32) * dsilu(a2)
    drb = drb + jnp.dot(da2, w2aT_ref[...], preferred_element_type=F32)

    denv = -0.5 * (jnp.pi / CUT) * jnp.sin(jnp.pi * rm / CUT) * mask
    drb_dr = (k * env * ((nvec * (jnp.pi / CUT)) * cth * rinv - sth * rinv * rinv)
              + k * sth * rinv * denv)
    dr = jnp.sum(drb * drb_dr, axis=1, keepdims=True)
    dv = dr * vec * rinv
    dv_ref[...] = dv
    ndv_ref[...] = -dv


def _tc_node1(at_ref, emb_ref, wm1_ref, h0_ref, hm1_ref):
    oh = (at_ref[...] == lax.broadcasted_iota(I32, (N, T), 1)).astype(F32)
    h0 = jnp.dot(oh, emb_ref[...], preferred_element_type=F32)
    h0_ref[...] = h0
    hm1_ref[...] = jnp.dot(h0, wm1_ref[...], preferred_element_type=F32)


def _tc_node2(aggp_ref, h0_ref, wu1_ref, wm2_ref, u1_ref, h1_ref, hm2_ref):
    agg = aggp_ref[0] + aggp_ref[1]
    u1 = jnp.dot(agg, wu1_ref[...], preferred_element_type=F32)
    h1 = h0_ref[...] + _silu(u1)
    u1_ref[...] = u1
    h1_ref[...] = h1
    hm2_ref[...] = jnp.dot(h1, wm2_ref[...], preferred_element_type=F32)


def _tc_node3(aggp_ref, h1_ref, wu2_ref, wout_ref, woutT_ref, wu2T_ref,
              e_ref, dg2_ref):
    agg = aggp_ref[0] + aggp_ref[1]
    u2 = jnp.dot(agg, wu2_ref[...], preferred_element_type=F32)
    s = jax.nn.sigmoid(u2)
    h2 = h1_ref[...] + u2 * s
    e_ref[0, 0] = jnp.sum(jnp.dot(h2, wout_ref[...], preferred_element_type=F32))
    du2 = woutT_ref[...] * (s * (1.0 + u2 * (1.0 - s)))
    dg2_ref[...] = jnp.dot(du2, wu2T_ref[...], preferred_element_type=F32)


def _tc_node4(dhmp_ref, u1_ref, wm2T_ref, wu1T_ref, woutT_ref, dg1_ref):
    dhm2 = dhmp_ref[0] + dhmp_ref[1]
    dh1 = woutT_ref[...] + jnp.dot(dhm2, wm2T_ref[...], preferred_element_type=F32)
    u1 = u1_ref[...]
    s = jax.nn.sigmoid(u1)
    du1 = dh1 * (s * (1.0 + u1 * (1.0 - s)))
    dg1_ref[...] = jnp.dot(du1, wu1T_ref[...], preferred_element_type=F32)


def _eblk(width):
    return pl.BlockSpec((BLK, width), lambda b: (b, 0))


def _full(shape):
    return pl.BlockSpec(shape, lambda b: tuple(0 for _ in shape))


def kernel(pos, emb, Wr1a, Wr1b, Wm1, Wu1, Wr2a, Wr2b, Wm2, Wu2, Wout,
           edge_index, atomic_numbers):
    j3 = edge_index[0].reshape(NW, NCH, CH)
    i3 = edge_index[1].reshape(NW, NCH, CH)
    pos16 = jnp.concatenate([pos, jnp.zeros((N, 13), F32)], axis=1)
    zND = jnp.zeros((N, D), F32)
    zN16 = jnp.zeros((N, 16), F32)
    at2 = atomic_numbers.reshape(N, 1)
    WoutT = Wout.T
    Wu1T, Wu2T, Wm2T = Wu1.T, Wu2.T, Wm2.T
    Wr1aT, Wr1bT, Wr2aT, Wr2bT = Wr1a.T, Wr1b.T, Wr2a.T, Wr2b.T

    pj4, pi4 = _sc_gather_pos(pos16, j3, i3)

    vec, w1, w2 = pl.pallas_call(
        _tc_radial_fwd,
        grid=(E // BLK,),
        in_specs=[_eblk(16), _eblk(16), _full((NB, 64)), _full((64, D)),
                  _full((NB, 64)), _full((64, D))],
        out_specs=[_eblk(16), _eblk(D), _eblk(D)],
        out_shape=[jax.ShapeDtypeStruct((E, 16), F32),
                   jax.ShapeDtypeStruct((E, D), F32),
                   jax.ShapeDtypeStruct((E, D), F32)],
    )(pj4.reshape(E, 16), pi4.reshape(E, 16), Wr1a, Wr1b, Wr2a, Wr2b)

    h0, hm1 = pl.pallas_call(
        _tc_node1,
        out_shape=[jax.ShapeDtypeStruct((N, D), F32),
                   jax.ShapeDtypeStruct((N, D), F32)],
    )(at2, emb, Wm1)

    agg1p = _sc_fwd_msg(w1.reshape(NW, NCH, CH, D), hm1, j3, i3, zND)

    u1, h1, hm2 = pl.pallas_call(
        _tc_node2,
        out_shape=[jax.ShapeDtypeStruct((N, D), F32)] * 3,
    )(agg1p, h0, Wu1, Wm2)

    agg2p = _sc_fwd_msg(w2.reshape(NW, NCH, CH, D), hm2, j3, i3, zND)

    e11, dg2 = pl.pallas_call(
        _tc_node3,
        out_shape=[jax.ShapeDtypeStruct((1, 1), F32),
                   jax.ShapeDtypeStruct((N, D), F32)],
    )(agg2p, h1, Wu2, Wout, WoutT, Wu2T)

    dw2_4, dhm2p = _sc_bwd2(w2.reshape(NW, NCH, CH, D), hm2, dg2, j3, i3, zND)

    dg1 = pl.pallas_call(
        _tc_node4,
        out_shape=jax.ShapeDtypeStruct((N, D), F32),
    )(dhm2p, u1, Wm2T, Wu1T, WoutT)

    dw1_4 = _sc_bwd1(hm1, dg1, j3, i3)

    dv, ndv = pl.pallas_call(
        _tc_radial_bwd,
        grid=(E // BLK,),
        in_specs=[_eblk(16), _eblk(D), _eblk(D), _full((NB, 64)),
                  _full((D, 64)), _full((64, NB)), _full((NB, 64)),
                  _full((D, 64)), _full((64, NB))],
        out_specs=[_eblk(16), _eblk(16)],
        out_shape=[jax.ShapeDtypeStruct((E, 16), F32),
                   jax.ShapeDtypeStruct((E, 16), F32)],
    )(vec, dw1_4.reshape(E, D), dw2_4.reshape(E, D), Wr1a, Wr1bT, Wr1aT,
      Wr2a, Wr2bT, Wr2aT)

    fp = _sc_force(ndv.reshape(NW, NCH, CH, 16), dv.reshape(NW, NCH, CH, 16),
                   j3, i3, zN16)

    forces = (fp[0] + fp[1])[:, :3]
    energy = e11.reshape(1)
    return (energy, forces)


# speedup vs baseline: 1.7521x; 1.7521x over previous
import functools

import jax
import jax.numpy as jnp
from jax import lax
from jax.experimental import pallas as pl
from jax.experimental.pallas import tpu as pltpu
from jax.experimental.pallas import tpu_sc as plsc

N = 10000
E = 320000
D = 128
T = 16
NB = 8
CUT = 4.0

NC = 2
NS = 16
NW = NC * NS
EPW = E // NW
CH = 80
NCH = EPW // CH

F32 = jnp.float32
I32 = jnp.int32

_MESH = plsc.VectorSubcoreMesh(
    core_axis_name="c", subcore_axis_name="s", num_cores=NC, num_subcores=NS)


def _worker_id():
    return lax.axis_index("s") * NC + lax.axis_index("c")


def _ew_mul(dst, a, b):
    def row(r, _):
        for rr in range(2):
            ri = r * 2 + rr
            for k in range(D // 16):
                sl = pl.ds(k * 16, 16)
                dst[ri, sl] = a[ri, sl] * b[ri, sl]
        return 0
    lax.fori_loop(0, CH // 2, row, 0)


@functools.partial(
    pl.kernel,
    out_type=jax.ShapeDtypeStruct((NW, NCH, CH, 4), F32),
    mesh=_MESH,
    scratch_types=[
        pltpu.VMEM((NCH, CH), I32),
        pltpu.VMEM((NCH, CH), I32),
        pltpu.VMEM((N,), F32),
        pltpu.VMEM((N,), F32),
        pltpu.VMEM((N,), F32),
        pltpu.VMEM((CH, 4), F32),
    ],
    compiler_params=pltpu.CompilerParams(needs_layout_passes=False),
)
def _sc_vec(px_hbm, py_hbm, pz_hbm, j_hbm, i_hbm, vec_hbm,
            jv, iv, px, py, pz, vbuf):
    w = _worker_id()
    pltpu.sync_copy(j_hbm.at[w], jv)
    pltpu.sync_copy(i_hbm.at[w], iv)
    pltpu.sync_copy(px_hbm, px)
    pltpu.sync_copy(py_hbm, py)
    pltpu.sync_copy(pz_hbm, pz)

    def body(c, _):
        for g in range(CH // 16):
            sl = pl.ds(g * 16, 16)
            j16 = jv[c, sl]
            i16 = iv[c, sl]
            e16 = lax.broadcasted_iota(I32, (16,), 0) + (g * 16)
            for comp, pref in ((0, px), (1, py), (2, pz)):
                vj = plsc.load_gather(pref, [j16])
                vi = plsc.load_gather(pref, [i16])
                cs = jnp.full((16,), comp, I32)
                plsc.store_scatter(vbuf, [e16, cs], vj - vi)
        pltpu.sync_copy(vbuf, vec_hbm.at[w, c])
        return 0
    lax.fori_loop(0, NCH, body, 0)


@functools.partial(
    pl.kernel,
    out_type=jax.ShapeDtypeStruct((NC, N, D), F32),
    mesh=_MESH,
    scratch_types=[
        pltpu.VMEM((1, CH), I32),
        pltpu.VMEM((1, CH), I32),
        pltpu.VMEM((CH, D), F32),
        pltpu.VMEM((CH, D), F32),
        pltpu.VMEM_SHARED((N, D), F32),
        pltpu.SemaphoreType.DMA,
        pltpu.SemaphoreType.DMA,
    ],
)
def _sc_fwd_msg(w_hbm, hm_hbm, j_hbm, i_hbm, z_hbm, out_hbm,
                jbuf, ibuf, wbuf, hmbuf, acc, s1, s2):
    cid = lax.axis_index("c")
    sid = lax.axis_index("s")
    w = sid * NC + cid

    @pl.when(sid == 0)
    def _():
        pltpu.sync_copy(z_hbm, acc)
    plsc.subcore_barrier()

    def body(c, _):
        pltpu.sync_copy(j_hbm.at[w, c], jbuf.at[0])
        pltpu.sync_copy(i_hbm.at[w, c], ibuf.at[0])
        cw = pltpu.async_copy(w_hbm.at[w, c], wbuf, s1)
        chm = pltpu.async_copy(hm_hbm.at[jbuf.at[0]], hmbuf, s2)
        cw.wait()
        chm.wait()
        _ew_mul(wbuf, wbuf, hmbuf)
        pltpu.sync_copy(wbuf, acc.at[ibuf.at[0]], add=True)
        return 0
    lax.fori_loop(0, NCH, body, 0)

    plsc.subcore_barrier()

    @pl.when(sid == 0)
    def _():
        pltpu.sync_copy(acc, out_hbm.at[cid])


@functools.partial(
    pl.kernel,
    out_type=(jax.ShapeDtypeStruct((NW, NCH, CH, D), F32),
              jax.ShapeDtypeStruct((NC, N, D), F32)),
    mesh=_MESH,
    scratch_types=[
        pltpu.VMEM((1, CH), I32),
        pltpu.VMEM((1, CH), I32),
        pltpu.VMEM((CH, D), F32),
        pltpu.VMEM((CH, D), F32),
        pltpu.VMEM((CH, D), F32),
        pltpu.VMEM_SHARED((N, D), F32),
        pltpu.SemaphoreType.DMA,
        pltpu.SemaphoreType.DMA,
        pltpu.SemaphoreType.DMA,
    ],
)
def _sc_bwd2(w_hbm, hm_hbm, dg_hbm, j_hbm, i_hbm, z_hbm, dw_hbm, out_hbm,
             jbuf, ibuf, wbuf, hmbuf, dgbuf, acc, s1, s2, s3):
    cid = lax.axis_index("c")
    sid = lax.axis_index("s")
    w = sid * NC + cid

    @pl.when(sid == 0)
    def _():
        pltpu.sync_copy(z_hbm, acc)
    plsc.subcore_barrier()

    def body(c, _):
        pltpu.sync_copy(j_hbm.at[w, c], jbuf.at[0])
        pltpu.sync_copy(i_hbm.at[w, c], ibuf.at[0])
        cw = pltpu.async_copy(w_hbm.at[w, c], wbuf, s1)
        chm = pltpu.async_copy(hm_hbm.at[jbuf.at[0]], hmbuf, s2)
        cdg = pltpu.async_copy(dg_hbm.at[ibuf.at[0]], dgbuf, s3)
        cw.wait()
        chm.wait()
        cdg.wait()
        _ew_mul(hmbuf, dgbuf, hmbuf)
        pltpu.sync_copy(hmbuf, dw_hbm.at[w, c])
        _ew_mul(dgbuf, dgbuf, wbuf)
        pltpu.sync_copy(dgbuf, acc.at[jbuf.at[0]], add=True)
        return 0
    lax.fori_loop(0, NCH, body, 0)

    plsc.subcore_barrier()

    @pl.when(sid == 0)
    def _():
        pltpu.sync_copy(acc, out_hbm.at[cid])


@functools.partial(
    pl.kernel,
    out_type=jax.ShapeDtypeStruct((NW, NCH, CH, D), F32),
    mesh=_MESH,
    scratch_types=[
        pltpu.VMEM((NCH, CH), I32),
        pltpu.VMEM((NCH, CH), I32),
        pltpu.VMEM((CH, D), F32),
        pltpu.VMEM((CH, D), F32),
        pltpu.VMEM((CH, D), F32),
        pltpu.SemaphoreType.DMA,
        pltpu.SemaphoreType.DMA,
    ],
)
def _sc_bwd1(hm_hbm, dg_hbm, j_hbm, i_hbm, dw_hbm,
             jv, iv, hmbuf, dgbuf, dwbuf, s1, s2):
    w = _worker_id()
    pltpu.sync_copy(j_hbm.at[w], jv)
    pltpu.sync_copy(i_hbm.at[w], iv)

    def body(c, _):
        chm = pltpu.async_copy(hm_hbm.at[jv.at[c]], hmbuf, s1)
        cdg = pltpu.async_copy(dg_hbm.at[iv.at[c]], dgbuf, s2)
        chm.wait()
        cdg.wait()
        _ew_mul(dwbuf, dgbuf, hmbuf)
        pltpu.sync_copy(dwbuf, dw_hbm.at[w, c])
        return 0
    lax.fori_loop(0, NCH, body, 0)


@functools.partial(
    pl.kernel,
    out_type=jax.ShapeDtypeStruct((NW, 3, N), F32),
    mesh=_MESH,
    scratch_types=[
        pltpu.VMEM((NCH, CH), I32),
        pltpu.VMEM((NCH, CH), I32),
        pltpu.VMEM((CH, 4), F32),
        pltpu.VMEM((1, N), F32),
        pltpu.VMEM((1, N), F32),
        pltpu.VMEM((1, N), F32),
    ],
    compiler_params=pltpu.CompilerParams(needs_layout_passes=False),
)
def _sc_force(dv_hbm, j_hbm, i_hbm, out_hbm, jv, iv, dvbuf, fx, fy, fz):
    w = _worker_id()
    pltpu.sync_copy(j_hbm.at[w], jv)
    pltpu.sync_copy(i_hbm.at[w], iv)

    def zero(t, _):
        z16 = jnp.zeros((16,), F32)
        sl = pl.ds(t * 16, 16)
        fx[0, sl] = z16
        fy[0, sl] = z16
        fz[0, sl] = z16
        return 0
    lax.fori_loop(0, N // 16, zero, 0)

    z16i = jnp.zeros((16,), I32)

    def body(c, _):
        pltpu.sync_copy(dv_hbm.at[w, c], dvbuf)
        for g in range(CH // 16):
            sl = pl.ds(g * 16, 16)
            j16 = jv[c, sl]
            i16 = iv[c, sl]
            e16 = lax.broadcasted_iota(I32, (16,), 0) + (g * 16)
            for comp, acc in ((0, fx), (1, fy), (2, fz)):
                cs = jnp.full((16,), comp, I32)
                v = plsc.load_gather(dvbuf, [e16, cs])
                plsc.addupdate_scatter(acc, [z16i, j16], -v)
                plsc.addupdate_scatter(acc, [z16i, i16], v)
        return 0
    lax.fori_loop(0, NCH, body, 0)

    pltpu.sync_copy(fx, out_hbm.at[w, pl.ds(0, 1)])
    pltpu.sync_copy(fy, out_hbm.at[w, pl.ds(1, 1)])
    pltpu.sync_copy(fz, out_hbm.at[w, pl.ds(2, 1)])


BLK = 3200


def _silu(x):
    return x * jax.nn.sigmoid(x)


def _radial_parts(vec):
    r2 = (vec[:, 0:1] * vec[:, 0:1] + vec[:, 1:2] * vec[:, 1:2]
          + vec[:, 2:3] * vec[:, 2:3] + 1e-6)
    r = jnp.sqrt(r2)
    rinv = 1.0 / r
    rm = jnp.minimum(r, CUT)
    mask = (r < CUT).astype(F32)
    env = 0.5 * (jnp.cos(jnp.pi * rm / CUT) + 1.0) * mask
    nvec = (lax.broadcasted_iota(I32, (1, NB), 1) + 1).astype(F32)
    theta = (jnp.pi / CUT) * r * nvec
    return r, rinv, rm, mask, env, nvec, theta


def _tc_radial_fwd(vec_ref, w1a_ref, w1b_ref, w2a_ref, w2b_ref,
                   w1_ref, w2_ref):
    r, rinv, rm, mask, env, nvec, theta = _radial_parts(vec_ref[...])
    k = jnp.sqrt(2.0 / CUT)
    rb = k * jnp.sin(theta) * rinv * env
    a1 = jnp.dot(rb, w1a_ref[...], preferred_element_type=F32)
    w1_ref[...] = jnp.dot(_silu(a1), w1b_ref[...], preferred_element_type=F32)
    a2 = jnp.dot(rb, w2a_ref[...], preferred_element_type=F32)
    w2_ref[...] = jnp.dot(_silu(a2), w2b_ref[...], preferred_element_type=F32)


def _tc_radial_bwd(vec_ref, dw1_ref, dw2_ref, w1a_ref, w1bT_ref, w1aT_ref,
                   w2a_ref, w2bT_ref, w2aT_ref, dv_ref):
    vec = vec_ref[...]
    r, rinv, rm, mask, env, nvec, theta = _radial_parts(vec)
    k = jnp.sqrt(2.0 / CUT)
    sth = jnp.sin(theta)
    cth = jnp.cos(theta)

    def dsilu(a):
        s = jax.nn.sigmoid(a)
        return s * (1.0 + a * (1.0 - s))

    rb = k * sth * rinv * env
    a1 = jnp.dot(rb, w1a_ref[...], preferred_element_type=F32)
    da1 = jnp.dot(dw1_ref[...], w1bT_ref[...], preferred_element_type=F32) * dsilu(a1)
    drb = jnp.dot(da1, w1aT_ref[...], preferred_element_type=F32)
    a2 = jnp.dot(rb, w2a_ref[...], preferred_element_type=F32)
    da2 = jnp.dot(dw2_ref[...], w2bT_ref[...], preferred_element_type=F32) * dsilu(a2)
    drb = drb + jnp.dot(da2, w2aT_ref[...], preferred_element_type=F32)

    denv = -0.5 * (jnp.pi / CUT) * jnp.sin(jnp.pi * rm / CUT) * mask
    drb_dr = (k * env * ((nvec * (jnp.pi / CUT)) * cth * rinv - sth * rinv * rinv)
              + k * sth * rinv * denv)
    dr = jnp.sum(drb * drb_dr, axis=1, keepdims=True)
    cmask = (lax.broadcasted_iota(I32, (1, 4), 1) < 3).astype(F32)
    dv_ref[...] = (dr * rinv * cmask) * vec


def _tc_node1(at_ref, emb_ref, wm1_ref, h0_ref, hm1_ref):
    oh = (at_ref[...] == lax.broadcasted_iota(I32, (N, T), 1)).astype(F32)
    h0 = jnp.dot(oh, emb_ref[...], preferred_element_type=F32)
    h0_ref[...] = h0
    hm1_ref[...] = jnp.dot(h0, wm1_ref[...], preferred_element_type=F32)


def _tc_node2(aggp_ref, h0_ref, wu1_ref, wm2_ref, u1_ref, h1_ref, hm2_ref):
    agg = aggp_ref[0] + aggp_ref[1]
    u1 = jnp.dot(agg, wu1_ref[...], preferred_element_type=F32)
    h1 = h0_ref[...] + _silu(u1)
    u1_ref[...] = u1
    h1_ref[...] = h1
    hm2_ref[...] = jnp.dot(h1, wm2_ref[...], preferred_element_type=F32)


def _tc_node3(aggp_ref, h1_ref, wu2_ref, wout_ref, woutT_ref, wu2T_ref,
              e_ref, dg2_ref):
    agg = aggp_ref[0] + aggp_ref[1]
    u2 = jnp.dot(agg, wu2_ref[...], preferred_element_type=F32)
    s = jax.nn.sigmoid(u2)
    h2 = h1_ref[...] + u2 * s
    e_ref[...] = jnp.sum(
        jnp.dot(h2, wout_ref[...], preferred_element_type=F32)).reshape(1, 1)
    du2 = woutT_ref[...] * (s * (1.0 + u2 * (1.0 - s)))
    dg2_ref[...] = jnp.dot(du2, wu2T_ref[...], preferred_element_type=F32)


def _tc_node4(dhmp_ref, u1_ref, wm2T_ref, wu1T_ref, woutT_ref, dg1_ref):
    dhm2 = dhmp_ref[0] + dhmp_ref[1]
    dh1 = woutT_ref[...] + jnp.dot(dhm2, wm2T_ref[...], preferred_element_type=F32)
    u1 = u1_ref[...]
    s = jax.nn.sigmoid(u1)
    du1 = dh1 * (s * (1.0 + u1 * (1.0 - s)))
    dg1_ref[...] = jnp.dot(du1, wu1T_ref[...], preferred_element_type=F32)


def _tc_fsum(fp_ref, out_ref):
    acc = fp_ref[0]
    for k in range(1, NW):
        acc = acc + fp_ref[k]
    out_ref[...] = acc


def _eblk(width):
    return pl.BlockSpec((BLK, width), lambda b: (b, 0))


def _full(shape):
    return pl.BlockSpec(shape, lambda b: tuple(0 for _ in shape))


def kernel(pos, emb, Wr1a, Wr1b, Wm1, Wu1, Wr2a, Wr2b, Wm2, Wu2, Wout,
           edge_index, atomic_numbers):
    j3 = edge_index[0].reshape(NW, NCH, CH)
    i3 = edge_index[1].reshape(NW, NCH, CH)
    zND = jnp.zeros((N, D), F32)
    at2 = atomic_numbers.reshape(N, 1)
    WoutT = Wout.T
    Wu1T, Wu2T, Wm2T = Wu1.T, Wu2.T, Wm2.T
    Wr1aT, Wr1bT, Wr2aT, Wr2bT = Wr1a.T, Wr1b.T, Wr2a.T, Wr2b.T

    vec4 = _sc_vec(pos[:, 0], pos[:, 1], pos[:, 2], j3, i3)

    w1, w2 = pl.pallas_call(
        _tc_radial_fwd,
        grid=(E // BLK,),
        in_specs=[_eblk(4), _full((NB, 64)), _full((64, D)),
                  _full((NB, 64)), _full((64, D))],
        out_specs=[_eblk(D), _eblk(D)],
        out_shape=[jax.ShapeDtypeStruct((E, D), F32),
                   jax.ShapeDtypeStruct((E, D), F32)],
    )(vec4.reshape(E, 4), Wr1a, Wr1b, Wr2a, Wr2b)

    h0, hm1 = pl.pallas_call(
        _tc_node1,
        out_shape=[jax.ShapeDtypeStruct((N, D), F32),
                   jax.ShapeDtypeStruct((N, D), F32)],
    )(at2, emb, Wm1)

    agg1p = _sc_fwd_msg(w1.reshape(NW, NCH, CH, D), hm1, j3, i3, zND)

    u1, h1, hm2 = pl.pallas_call(
        _tc_node2,
        out_shape=[jax.ShapeDtypeStruct((N, D), F32)] * 3,
    )(agg1p, h0, Wu1, Wm2)

    agg2p = _sc_fwd_msg(w2.reshape(NW, NCH, CH, D), hm2, j3, i3, zND)

    e11, dg2 = pl.pallas_call(
        _tc_node3,
        out_shape=[jax.ShapeDtypeStruct((1, 1), F32),
                   jax.ShapeDtypeStruct((N, D), F32)],
    )(agg2p, h1, Wu2, Wout, WoutT, Wu2T)

    dw2_4, dhm2p = _sc_bwd2(w2.reshape(NW, NCH, CH, D), hm2, dg2, j3, i3, zND)

    dg1 = pl.pallas_call(
        _tc_node4,
        out_shape=jax.ShapeDtypeStruct((N, D), F32),
    )(dhm2p, u1, Wm2T, Wu1T, WoutT)

    dw1_4 = _sc_bwd1(hm1, dg1, j3, i3)

    dv = pl.pallas_call(
        _tc_radial_bwd,
        grid=(E // BLK,),
        in_specs=[_eblk(4), _eblk(D), _eblk(D), _full((NB, 64)),
                  _full((D, 64)), _full((64, NB)), _full((NB, 64)),
                  _full((D, 64)), _full((64, NB))],
        out_specs=[_eblk(4)],
        out_shape=[jax.ShapeDtypeStruct((E, 4), F32)],
    )(vec4.reshape(E, 4), dw1_4.reshape(E, D), dw2_4.reshape(E, D),
      Wr1a, Wr1bT, Wr1aT, Wr2a, Wr2bT, Wr2aT)[0]

    fp = _sc_force(dv.reshape(NW, NCH, CH, 4), j3, i3)

    fsum = pl.pallas_call(
        _tc_fsum,
        out_shape=jax.ShapeDtypeStruct((3, N), F32),
    )(fp)

    forces = fsum.T
    energy = e11.reshape(1)
    return (energy, forces)


# speedup vs baseline: 1.7777x; 1.0146x over previous
import functools

import jax
import jax.numpy as jnp
from jax import lax
from jax.experimental import pallas as pl
from jax.experimental.pallas import tpu as pltpu
from jax.experimental.pallas import tpu_sc as plsc

N = 10000
E = 320000
D = 128
T = 16
NB = 8
CUT = 4.0

NC = 2
NS = 16
NW = NC * NS
EPW = E // NW
CH = 80
NCH = EPW // CH

F32 = jnp.float32
I32 = jnp.int32

_MESH = plsc.VectorSubcoreMesh(
    core_axis_name="c", subcore_axis_name="s", num_cores=NC, num_subcores=NS)


def _worker_id():
    return lax.axis_index("s") * NC + lax.axis_index("c")


def _ew_mul(dst, a, b):
    def row(r, _):
        for rr in range(2):
            ri = r * 2 + rr
            for k in range(D // 16):
                sl = pl.ds(k * 16, 16)
                dst[ri, sl] = a[ri, sl] * b[ri, sl]
        return 0
    lax.fori_loop(0, CH // 2, row, 0)


@functools.partial(
    pl.kernel,
    out_type=jax.ShapeDtypeStruct((NW, NCH, CH, 4), F32),
    mesh=_MESH,
    scratch_types=[
        pltpu.VMEM((NCH, CH), I32),
        pltpu.VMEM((NCH, CH), I32),
        pltpu.VMEM((N,), F32),
        pltpu.VMEM((N,), F32),
        pltpu.VMEM((N,), F32),
        pltpu.VMEM((CH, 4), F32),
    ],
    compiler_params=pltpu.CompilerParams(needs_layout_passes=False),
)
def _sc_vec(px_hbm, py_hbm, pz_hbm, j_hbm, i_hbm, vec_hbm,
            jv, iv, px, py, pz, vbuf):
    w = _worker_id()
    pltpu.sync_copy(j_hbm.at[w], jv)
    pltpu.sync_copy(i_hbm.at[w], iv)
    pltpu.sync_copy(px_hbm, px)
    pltpu.sync_copy(py_hbm, py)
    pltpu.sync_copy(pz_hbm, pz)

    def body(c, _):
        for g in range(CH // 16):
            sl = pl.ds(g * 16, 16)
            j16 = jv[c, sl]
            i16 = iv[c, sl]
            e16 = lax.broadcasted_iota(I32, (16,), 0) + (g * 16)
            for comp, pref in ((0, px), (1, py), (2, pz)):
                vj = plsc.load_gather(pref, [j16])
                vi = plsc.load_gather(pref, [i16])
                cs = jnp.full((16,), comp, I32)
                plsc.store_scatter(vbuf, [e16, cs], vj - vi)
        pltpu.sync_copy(vbuf, vec_hbm.at[w, c])
        return 0
    lax.fori_loop(0, NCH, body, 0)


@functools.partial(
    pl.kernel,
    out_type=jax.ShapeDtypeStruct((NC, N, D), F32),
    mesh=_MESH,
    scratch_types=[
        pltpu.VMEM((1, CH), I32),
        pltpu.VMEM((1, CH), I32),
        pltpu.VMEM((CH, D), F32),
        pltpu.VMEM((CH, D), F32),
        pltpu.VMEM_SHARED((N, D), F32),
        pltpu.SemaphoreType.DMA,
        pltpu.SemaphoreType.DMA,
    ],
)
def _sc_fwd_msg(w_hbm, hm_hbm, j_hbm, i_hbm, z_hbm, out_hbm,
                jbuf, ibuf, wbuf, hmbuf, acc, s1, s2):
    cid = lax.axis_index("c")
    sid = lax.axis_index("s")
    w = sid * NC + cid

    @pl.when(sid == 0)
    def _():
        pltpu.sync_copy(z_hbm, acc)
    plsc.subcore_barrier()

    def body(c, _):
        pltpu.sync_copy(j_hbm.at[w, c], jbuf.at[0])
        pltpu.sync_copy(i_hbm.at[w, c], ibuf.at[0])
        cw = pltpu.async_copy(w_hbm.at[w, c], wbuf, s1)
        chm = pltpu.async_copy(hm_hbm.at[jbuf.at[0]], hmbuf, s2)
        cw.wait()
        chm.wait()
        _ew_mul(wbuf, wbuf, hmbuf)
        pltpu.sync_copy(wbuf, acc.at[ibuf.at[0]], add=True)
        return 0
    lax.fori_loop(0, NCH, body, 0)

    plsc.subcore_barrier()

    @pl.when(sid == 0)
    def _():
        pltpu.sync_copy(acc, out_hbm.at[cid])


@functools.partial(
    pl.kernel,
    out_type=(jax.ShapeDtypeStruct((NW, NCH, CH, D), F32),
              jax.ShapeDtypeStruct((NC, N, D), F32)),
    mesh=_MESH,
    scratch_types=[
        pltpu.VMEM((1, CH), I32),
        pltpu.VMEM((1, CH), I32),
        pltpu.VMEM((CH, D), F32),
        pltpu.VMEM((CH, D), F32),
        pltpu.VMEM((CH, D), F32),
        pltpu.VMEM_SHARED((N, D), F32),
        pltpu.SemaphoreType.DMA,
        pltpu.SemaphoreType.DMA,
        pltpu.SemaphoreType.DMA,
    ],
)
def _sc_bwd2(w_hbm, hm_hbm, dg_hbm, j_hbm, i_hbm, z_hbm, dw_hbm, out_hbm,
             jbuf, ibuf, wbuf, hmbuf, dgbuf, acc, s1, s2, s3):
    cid = lax.axis_index("c")
    sid = lax.axis_index("s")
    w = sid * NC + cid

    @pl.when(sid == 0)
    def _():
        pltpu.sync_copy(z_hbm, acc)
    plsc.subcore_barrier()

    def body(c, _):
        pltpu.sync_copy(j_hbm.at[w, c], jbuf.at[0])
        pltpu.sync_copy(i_hbm.at[w, c], ibuf.at[0])
        cw = pltpu.async_copy(w_hbm.at[w, c], wbuf, s1)
        chm = pltpu.async_copy(hm_hbm.at[jbuf.at[0]], hmbuf, s2)
        cdg = pltpu.async_copy(dg_hbm.at[ibuf.at[0]], dgbuf, s3)
        cw.wait()
        chm.wait()
        cdg.wait()
        _ew_mul(hmbuf, dgbuf, hmbuf)
        pltpu.sync_copy(hmbuf, dw_hbm.at[w, c])
        _ew_mul(dgbuf, dgbuf, wbuf)
        pltpu.sync_copy(dgbuf, acc.at[jbuf.at[0]], add=True)
        return 0
    lax.fori_loop(0, NCH, body, 0)

    plsc.subcore_barrier()

    @pl.when(sid == 0)
    def _():
        pltpu.sync_copy(acc, out_hbm.at[cid])


@functools.partial(
    pl.kernel,
    out_type=jax.ShapeDtypeStruct((NW, NCH, CH, D), F32),
    mesh=_MESH,
    scratch_types=[
        pltpu.VMEM((NCH, CH), I32),
        pltpu.VMEM((NCH, CH), I32),
        pltpu.VMEM((CH, D), F32),
        pltpu.VMEM((CH, D), F32),
        pltpu.VMEM((CH, D), F32),
        pltpu.SemaphoreType.DMA,
        pltpu.SemaphoreType.DMA,
    ],
)
def _sc_bwd1(hm_hbm, dg_hbm, j_hbm, i_hbm, dw_hbm,
             jv, iv, hmbuf, dgbuf, dwbuf, s1, s2):
    w = _worker_id()
    pltpu.sync_copy(j_hbm.at[w], jv)
    pltpu.sync_copy(i_hbm.at[w], iv)

    def body(c, _):
        chm = pltpu.async_copy(hm_hbm.at[jv.at[c]], hmbuf, s1)
        cdg = pltpu.async_copy(dg_hbm.at[iv.at[c]], dgbuf, s2)
        chm.wait()
        cdg.wait()
        _ew_mul(dwbuf, dgbuf, hmbuf)
        pltpu.sync_copy(dwbuf, dw_hbm.at[w, c])
        return 0
    lax.fori_loop(0, NCH, body, 0)


@functools.partial(
    pl.kernel,
    out_type=jax.ShapeDtypeStruct((NW, 3, N), F32),
    mesh=_MESH,
    scratch_types=[
        pltpu.VMEM((NCH, CH), I32),
        pltpu.VMEM((NCH, CH), I32),
        pltpu.VMEM((CH, 4), F32),
        pltpu.VMEM((1, N), F32),
        pltpu.VMEM((1, N), F32),
        pltpu.VMEM((1, N), F32),
    ],
    compiler_params=pltpu.CompilerParams(needs_layout_passes=False),
)
def _sc_force(dv_hbm, j_hbm, i_hbm, out_hbm, jv, iv, dvbuf, fx, fy, fz):
    w = _worker_id()
    pltpu.sync_copy(j_hbm.at[w], jv)
    pltpu.sync_copy(i_hbm.at[w], iv)

    def zero(t, _):
        z16 = jnp.zeros((16,), F32)
        sl = pl.ds(t * 16, 16)
        fx[0, sl] = z16
        fy[0, sl] = z16
        fz[0, sl] = z16
        return 0
    lax.fori_loop(0, N // 16, zero, 0)

    z16i = jnp.zeros((16,), I32)

    def body(c, _):
        pltpu.sync_copy(dv_hbm.at[w, c], dvbuf)
        for g in range(CH // 16):
            sl = pl.ds(g * 16, 16)
            j16 = jv[c, sl]
            i16 = iv[c, sl]
            e16 = lax.broadcasted_iota(I32, (16,), 0) + (g * 16)
            for comp, acc in ((0, fx), (1, fy), (2, fz)):
                cs = jnp.full((16,), comp, I32)
                v = plsc.load_gather(dvbuf, [e16, cs])
                plsc.addupdate_scatter(acc, [z16i, j16], -v)
                plsc.addupdate_scatter(acc, [z16i, i16], v)
        return 0
    lax.fori_loop(0, NCH, body, 0)

    pltpu.sync_copy(fx, out_hbm.at[w, pl.ds(0, 1)])
    pltpu.sync_copy(fy, out_hbm.at[w, pl.ds(1, 1)])
    pltpu.sync_copy(fz, out_hbm.at[w, pl.ds(2, 1)])


BLK = 3200


def _silu(x):
    return x * jax.nn.sigmoid(x)


def _radial_parts(vec):
    r2 = (vec[:, 0:1] * vec[:, 0:1] + vec[:, 1:2] * vec[:, 1:2]
          + vec[:, 2:3] * vec[:, 2:3] + 1e-6)
    r = jnp.sqrt(r2)
    rinv = 1.0 / r
    rm = jnp.minimum(r, CUT)
    mask = (r < CUT).astype(F32)
    env = 0.5 * (jnp.cos(jnp.pi * rm / CUT) + 1.0) * mask
    nvec = (lax.broadcasted_iota(I32, (1, NB), 1) + 1).astype(F32)
    theta = (jnp.pi / CUT) * r * nvec
    return r, rinv, rm, mask, env, nvec, theta


def _tc_radial_fwd(vec_ref, wa12_ref, wbd_ref, w1_ref, w2_ref):
    r, rinv, rm, mask, env, nvec, theta = _radial_parts(vec_ref[...])
    k = jnp.sqrt(2.0 / CUT)
    rb = k * jnp.sin(theta) * rinv * env
    a12 = jnp.dot(rb, wa12_ref[...], preferred_element_type=F32)
    w12 = jnp.dot(_silu(a12), wbd_ref[...], preferred_element_type=F32)
    w1_ref[...] = w12[:, :D]
    w2_ref[...] = w12[:, D:]


def _tc_radial_bwd(vec_ref, dw1_ref, dw2_ref, wa12_ref, wbdT_ref, wa12T_ref,
                   dv_ref):
    vec = vec_ref[...]
    r, rinv, rm, mask, env, nvec, theta = _radial_parts(vec)
    k = jnp.sqrt(2.0 / CUT)
    sth = jnp.sin(theta)
    cth = jnp.cos(theta)

    def dsilu(a):
        s = jax.nn.sigmoid(a)
        return s * (1.0 + a * (1.0 - s))

    rb = k * sth * rinv * env
    a12 = jnp.dot(rb, wa12_ref[...], preferred_element_type=F32)
    dw12 = jnp.concatenate([dw1_ref[...], dw2_ref[...]], axis=1)
    ds12 = jnp.dot(dw12, wbdT_ref[...], preferred_element_type=F32)
    da12 = ds12 * dsilu(a12)
    drb = jnp.dot(da12, wa12T_ref[...], preferred_element_type=F32)

    denv = -0.5 * (jnp.pi / CUT) * jnp.sin(jnp.pi * rm / CUT) * mask
    drb_dr = (k * env * ((nvec * (jnp.pi / CUT)) * cth * rinv - sth * rinv * rinv)
              + k * sth * rinv * denv)
    dr = jnp.sum(drb * drb_dr, axis=1, keepdims=True)
    cmask = (lax.broadcasted_iota(I32, (1, 4), 1) < 3).astype(F32)
    dv_ref[...] = (dr * rinv * cmask) * vec


def _tc_node1(at_ref, emb_ref, wm1_ref, h0_ref, hm1_ref):
    oh = (at_ref[...] == lax.broadcasted_iota(I32, (N, T), 1)).astype(F32)
    h0 = jnp.dot(oh, emb_ref[...], preferred_element_type=F32)
    h0_ref[...] = h0
    hm1_ref[...] = jnp.dot(h0, wm1_ref[...], preferred_element_type=F32)


def _tc_node2(aggp_ref, h0_ref, wu1_ref, wm2_ref, u1_ref, h1_ref, hm2_ref):
    agg = aggp_ref[0] + aggp_ref[1]
    u1 = jnp.dot(agg, wu1_ref[...], preferred_element_type=F32)
    h1 = h0_ref[...] + _silu(u1)
    u1_ref[...] = u1
    h1_ref[...] = h1
    hm2_ref[...] = jnp.dot(h1, wm2_ref[...], preferred_element_type=F32)


def _tc_node3(aggp_ref, h1_ref, wu2_ref, wout_ref, woutT_ref, wu2T_ref,
              e_ref, dg2_ref):
    agg = aggp_ref[0] + aggp_ref[1]
    u2 = jnp.dot(agg, wu2_ref[...], preferred_element_type=F32)
    s = jax.nn.sigmoid(u2)
    h2 = h1_ref[...] + u2 * s
    e_ref[...] = jnp.sum(
        jnp.dot(h2, wout_ref[...], preferred_element_type=F32)).reshape(1, 1)
    du2 = woutT_ref[...] * (s * (1.0 + u2 * (1.0 - s)))
    dg2_ref[...] = jnp.dot(du2, wu2T_ref[...], preferred_element_type=F32)


def _tc_node4(dhmp_ref, u1_ref, wm2T_ref, wu1T_ref, woutT_ref, dg1_ref):
    dhm2 = dhmp_ref[0] + dhmp_ref[1]
    dh1 = woutT_ref[...] + jnp.dot(dhm2, wm2T_ref[...], preferred_element_type=F32)
    u1 = u1_ref[...]
    s = jax.nn.sigmoid(u1)
    du1 = dh1 * (s * (1.0 + u1 * (1.0 - s)))
    dg1_ref[...] = jnp.dot(du1, wu1T_ref[...], preferred_element_type=F32)


def _tc_fsum(fp_ref, out_ref):
    acc = fp_ref[0]
    for k in range(1, NW):
        acc = acc + fp_ref[k]
    out_ref[...] = acc


def _eblk(width):
    return pl.BlockSpec((BLK, width), lambda b: (b, 0))


def _full(shape):
    return pl.BlockSpec(shape, lambda b: tuple(0 for _ in shape))


def kernel(pos, emb, Wr1a, Wr1b, Wm1, Wu1, Wr2a, Wr2b, Wm2, Wu2, Wout,
           edge_index, atomic_numbers):
    j3 = edge_index[0].reshape(NW, NCH, CH)
    i3 = edge_index[1].reshape(NW, NCH, CH)
    zND = jnp.zeros((N, D), F32)
    at2 = atomic_numbers.reshape(N, 1)
    WoutT = Wout.T
    Wu1T, Wu2T, Wm2T = Wu1.T, Wu2.T, Wm2.T
    Wa12 = jnp.concatenate([Wr1a, Wr2a], axis=1)
    z64 = jnp.zeros((64, D), F32)
    Wbd = jnp.concatenate(
        [jnp.concatenate([Wr1b, z64], axis=0),
         jnp.concatenate([z64, Wr2b], axis=0)], axis=1)
    WbdT = Wbd.T
    Wa12T = Wa12.T

    vec4 = _sc_vec(pos[:, 0], pos[:, 1], pos[:, 2], j3, i3)

    w1, w2 = pl.pallas_call(
        _tc_radial_fwd,
        grid=(E // BLK,),
        in_specs=[_eblk(4), _full((NB, D)), _full((D, 2 * D))],
        out_specs=[_eblk(D), _eblk(D)],
        out_shape=[jax.ShapeDtypeStruct((E, D), F32),
                   jax.ShapeDtypeStruct((E, D), F32)],
    )(vec4.reshape(E, 4), Wa12, Wbd)

    h0, hm1 = pl.pallas_call(
        _tc_node1,
        out_shape=[jax.ShapeDtypeStruct((N, D), F32),
                   jax.ShapeDtypeStruct((N, D), F32)],
    )(at2, emb, Wm1)

    agg1p = _sc_fwd_msg(w1.reshape(NW, NCH, CH, D), hm1, j3, i3, zND)

    u1, h1, hm2 = pl.pallas_call(
        _tc_node2,
        out_shape=[jax.ShapeDtypeStruct((N, D), F32)] * 3,
    )(agg1p, h0, Wu1, Wm2)

    agg2p = _sc_fwd_msg(w2.reshape(NW, NCH, CH, D), hm2, j3, i3, zND)

    e11, dg2 = pl.pallas_call(
        _tc_node3,
        out_shape=[jax.ShapeDtypeStruct((1, 1), F32),
                   jax.ShapeDtypeStruct((N, D), F32)],
    )(agg2p, h1, Wu2, Wout, WoutT, Wu2T)

    dw2_4, dhm2p = _sc_bwd2(w2.reshape(NW, NCH, CH, D), hm2, dg2, j3, i3, zND)

    dg1 = pl.pallas_call(
        _tc_node4,
        out_shape=jax.ShapeDtypeStruct((N, D), F32),
    )(dhm2p, u1, Wm2T, Wu1T, WoutT)

    dw1_4 = _sc_bwd1(hm1, dg1, j3, i3)

    dv = pl.pallas_call(
        _tc_radial_bwd,
        grid=(E // BLK,),
        in_specs=[_eblk(4), _eblk(D), _eblk(D), _full((NB, D)),
                  _full((2 * D, D)), _full((D, NB))],
        out_specs=[_eblk(4)],
        out_shape=[jax.ShapeDtypeStruct((E, 4), F32)],
    )(vec4.reshape(E, 4), dw1_4.reshape(E, D), dw2_4.reshape(E, D),
      Wa12, WbdT, Wa12T)[0]

    fp = _sc_force(dv.reshape(NW, NCH, CH, 4), j3, i3)

    fsum = pl.pallas_call(
        _tc_fsum,
        out_shape=jax.ShapeDtypeStruct((3, N), F32),
    )(fp)

    forces = fsum.T
    energy = e11.reshape(1)
    return (energy, forces)


# speedup vs baseline: 3.3598x; 1.8899x over previous
import functools

import jax
import jax.numpy as jnp
from jax import lax
from jax.experimental import pallas as pl
from jax.experimental.pallas import tpu as pltpu
from jax.experimental.pallas import tpu_sc as plsc

N = 10000
E = 320000
D = 128
T = 16
NB = 8
CUT = 4.0

NC = 2
NS = 16
NW = NC * NS
EPW = E // NW
CH = 80
NCH = EPW // CH

F32 = jnp.float32
I32 = jnp.int32

_MESH = plsc.VectorSubcoreMesh(
    core_axis_name="c", subcore_axis_name="s", num_cores=NC, num_subcores=NS)


def _worker_id():
    return lax.axis_index("s") * NC + lax.axis_index("c")


def _ew_mul(dst, a, b):
    def row(r, _):
        for rr in range(2):
            ri = r * 2 + rr
            for k in range(D // 16):
                sl = pl.ds(k * 16, 16)
                dst[ri, sl] = a[ri, sl] * b[ri, sl]
        return 0
    lax.fori_loop(0, CH // 2, row, 0)


@functools.partial(
    pl.kernel,
    out_type=jax.ShapeDtypeStruct((NW, NCH, CH, 4), F32),
    mesh=_MESH,
    scratch_types=[
        pltpu.VMEM((NCH, CH), I32),
        pltpu.VMEM((NCH, CH), I32),
        pltpu.VMEM((N,), F32),
        pltpu.VMEM((N,), F32),
        pltpu.VMEM((N,), F32),
        pltpu.VMEM((CH, 4), F32),
    ],
    compiler_params=pltpu.CompilerParams(needs_layout_passes=False),
)
def _sc_vec(px_hbm, py_hbm, pz_hbm, j_hbm, i_hbm, vec_hbm,
            jv, iv, px, py, pz, vbuf):
    w = _worker_id()
    pltpu.sync_copy(j_hbm.at[w], jv)
    pltpu.sync_copy(i_hbm.at[w], iv)
    pltpu.sync_copy(px_hbm, px)
    pltpu.sync_copy(py_hbm, py)
    pltpu.sync_copy(pz_hbm, pz)

    def body(c, _):
        for g in range(CH // 16):
            sl = pl.ds(g * 16, 16)
            j16 = jv[c, sl]
            i16 = iv[c, sl]
            e16 = lax.broadcasted_iota(I32, (16,), 0) + (g * 16)
            for comp, pref in ((0, px), (1, py), (2, pz)):
                vj = plsc.load_gather(pref, [j16])
                vi = plsc.load_gather(pref, [i16])
                cs = jnp.full((16,), comp, I32)
                plsc.store_scatter(vbuf, [e16, cs], vj - vi)
        pltpu.sync_copy(vbuf, vec_hbm.at[w, c])
        return 0
    lax.fori_loop(0, NCH, body, 0)


@functools.partial(
    pl.kernel,
    out_type=jax.ShapeDtypeStruct((NC, N, D), F32),
    mesh=_MESH,
    scratch_types=[
        pltpu.VMEM((1, CH), I32),
        pltpu.VMEM((1, CH), I32),
        pltpu.VMEM((CH, D), F32),
        pltpu.VMEM((CH, D), F32),
        pltpu.VMEM_SHARED((N, D), F32),
        pltpu.SemaphoreType.DMA,
        pltpu.SemaphoreType.DMA,
    ],
)
def _sc_fwd_msg(w_hbm, hm_hbm, j_hbm, i_hbm, z_hbm, out_hbm,
                jbuf, ibuf, wbuf, hmbuf, acc, s1, s2):
    cid = lax.axis_index("c")
    sid = lax.axis_index("s")
    w = sid * NC + cid

    @pl.when(sid == 0)
    def _():
        pltpu.sync_copy(z_hbm, acc)
    plsc.subcore_barrier()

    def body(c, _):
        pltpu.sync_copy(j_hbm.at[w, c], jbuf.at[0])
        pltpu.sync_copy(i_hbm.at[w, c], ibuf.at[0])
        cw = pltpu.async_copy(w_hbm.at[w, c], wbuf, s1)
        chm = pltpu.async_copy(hm_hbm.at[jbuf.at[0]], hmbuf, s2)
        cw.wait()
        chm.wait()
        _ew_mul(wbuf, wbuf, hmbuf)
        pltpu.sync_copy(wbuf, acc.at[ibuf.at[0]], add=True)
        return 0
    lax.fori_loop(0, NCH, body, 0)

    plsc.subcore_barrier()

    @pl.when(sid == 0)
    def _():
        pltpu.sync_copy(acc, out_hbm.at[cid])


@functools.partial(
    pl.kernel,
    out_type=(jax.ShapeDtypeStruct((NW, NCH, CH, D), F32),
              jax.ShapeDtypeStruct((NC, N, D), F32)),
    mesh=_MESH,
    scratch_types=[
        pltpu.VMEM((1, CH), I32),
        pltpu.VMEM((1, CH), I32),
        pltpu.VMEM((CH, D), F32),
        pltpu.VMEM((CH, D), F32),
        pltpu.VMEM((CH, D), F32),
        pltpu.VMEM_SHARED((N, D), F32),
        pltpu.SemaphoreType.DMA,
        pltpu.SemaphoreType.DMA,
        pltpu.SemaphoreType.DMA,
    ],
)
def _sc_bwd2(w_hbm, hm_hbm, dg_hbm, j_hbm, i_hbm, z_hbm, dw_hbm, out_hbm,
             jbuf, ibuf, wbuf, hmbuf, dgbuf, acc, s1, s2, s3):
    cid = lax.axis_index("c")
    sid = lax.axis_index("s")
    w = sid * NC + cid

    @pl.when(sid == 0)
    def _():
        pltpu.sync_copy(z_hbm, acc)
    plsc.subcore_barrier()

    def body(c, _):
        pltpu.sync_copy(j_hbm.at[w, c], jbuf.at[0])
        pltpu.sync_copy(i_hbm.at[w, c], ibuf.at[0])
        cw = pltpu.async_copy(w_hbm.at[w, c], wbuf, s1)
        chm = pltpu.async_copy(hm_hbm.at[jbuf.at[0]], hmbuf, s2)
        cdg = pltpu.async_copy(dg_hbm.at[ibuf.at[0]], dgbuf, s3)
        cw.wait()
        chm.wait()
        cdg.wait()
        _ew_mul(hmbuf, dgbuf, hmbuf)
        pltpu.sync_copy(hmbuf, dw_hbm.at[w, c])
        _ew_mul(dgbuf, dgbuf, wbuf)
        pltpu.sync_copy(dgbuf, acc.at[jbuf.at[0]], add=True)
        return 0
    lax.fori_loop(0, NCH, body, 0)

    plsc.subcore_barrier()

    @pl.when(sid == 0)
    def _():
        pltpu.sync_copy(acc, out_hbm.at[cid])


@functools.partial(
    pl.kernel,
    out_type=jax.ShapeDtypeStruct((NW, NCH, CH, D), F32),
    mesh=_MESH,
    scratch_types=[
        pltpu.VMEM((NCH, CH), I32),
        pltpu.VMEM((NCH, CH), I32),
        pltpu.VMEM((CH, D), F32),
        pltpu.VMEM((CH, D), F32),
        pltpu.VMEM((CH, D), F32),
        pltpu.SemaphoreType.DMA,
        pltpu.SemaphoreType.DMA,
    ],
)
def _sc_bwd1(hm_hbm, dg_hbm, j_hbm, i_hbm, dw_hbm,
             jv, iv, hmbuf, dgbuf, dwbuf, s1, s2):
    w = _worker_id()
    pltpu.sync_copy(j_hbm.at[w], jv)
    pltpu.sync_copy(i_hbm.at[w], iv)

    def body(c, _):
        chm = pltpu.async_copy(hm_hbm.at[jv.at[c]], hmbuf, s1)
        cdg = pltpu.async_copy(dg_hbm.at[iv.at[c]], dgbuf, s2)
        chm.wait()
        cdg.wait()
        _ew_mul(dwbuf, dgbuf, hmbuf)
        pltpu.sync_copy(dwbuf, dw_hbm.at[w, c])
        return 0
    lax.fori_loop(0, NCH, body, 0)


@functools.partial(
    pl.kernel,
    out_type=jax.ShapeDtypeStruct((NW, 3, N), F32),
    mesh=_MESH,
    scratch_types=[
        pltpu.VMEM((NCH, CH), I32),
        pltpu.VMEM((NCH, CH), I32),
        pltpu.VMEM((CH, 4), F32),
        pltpu.VMEM((1, N), F32),
        pltpu.VMEM((1, N), F32),
        pltpu.VMEM((1, N), F32),
    ],
    compiler_params=pltpu.CompilerParams(needs_layout_passes=False),
)
def _sc_force(dv_hbm, j_hbm, i_hbm, out_hbm, jv, iv, dvbuf, fx, fy, fz):
    w = _worker_id()
    pltpu.sync_copy(j_hbm.at[w], jv)
    pltpu.sync_copy(i_hbm.at[w], iv)

    def zero(t, _):
        z16 = jnp.zeros((16,), F32)
        sl = pl.ds(t * 16, 16)
        fx[0, sl] = z16
        fy[0, sl] = z16
        fz[0, sl] = z16
        return 0
    lax.fori_loop(0, N // 16, zero, 0)

    z16i = jnp.zeros((16,), I32)

    def body(c, _):
        pltpu.sync_copy(dv_hbm.at[w, c], dvbuf)
        for g in range(CH // 16):
            sl = pl.ds(g * 16, 16)
            j16 = jv[c, sl]
            i16 = iv[c, sl]
            e16 = lax.broadcasted_iota(I32, (16,), 0) + (g * 16)
            for comp, acc in ((0, fx), (1, fy), (2, fz)):
                cs = jnp.full((16,), comp, I32)
                v = plsc.load_gather(dvbuf, [e16, cs])
                plsc.addupdate_scatter(acc, [z16i, j16], -v)
                plsc.addupdate_scatter(acc, [z16i, i16], v)
        return 0
    lax.fori_loop(0, NCH, body, 0)

    pltpu.sync_copy(fx, out_hbm.at[w, pl.ds(0, 1)])
    pltpu.sync_copy(fy, out_hbm.at[w, pl.ds(1, 1)])
    pltpu.sync_copy(fz, out_hbm.at[w, pl.ds(2, 1)])


BLK = 3200


def _silu(x):
    return x * jax.nn.sigmoid(x)


def _radial_parts_t(vt):
    vx, vy, vz = vt[0:1], vt[1:2], vt[2:3]
    r2 = vx * vx + vy * vy + vz * vz + 1e-6
    r = jnp.sqrt(r2)
    rinv = 1.0 / r
    rm = jnp.minimum(r, CUT)
    mask = (r < CUT).astype(F32)
    env = 0.5 * (jnp.cos(jnp.pi * rm / CUT) + 1.0) * mask
    nvec = (lax.broadcasted_iota(I32, (NB, 1), 0) + 1).astype(F32)
    theta = (jnp.pi / CUT) * r * nvec
    return vx, vy, vz, r, rinv, rm, mask, env, nvec, theta


def _tc_radial_fwd(vt_ref, wa12_ref, wbd_ref, w1_ref, w2_ref):
    vx, vy, vz, r, rinv, rm, mask, env, nvec, theta = _radial_parts_t(vt_ref[...])
    k = jnp.sqrt(2.0 / CUT)
    rbt = k * jnp.sin(theta) * (rinv * env)
    a12 = lax.dot_general(rbt, wa12_ref[...], (((0,), (0,)), ((), ())),
                          preferred_element_type=F32)
    w12 = jnp.dot(_silu(a12), wbd_ref[...], preferred_element_type=F32)
    w1_ref[...] = w12[:, :D]
    w2_ref[...] = w12[:, D:]


def _tc_radial_bwd(vt_ref, dw1_ref, dw2_ref, wa12_ref, wbdT_ref, wa12T_ref,
                   dvt_ref):
    vx, vy, vz, r, rinv, rm, mask, env, nvec, theta = _radial_parts_t(vt_ref[...])
    k = jnp.sqrt(2.0 / CUT)
    sth = jnp.sin(theta)
    cth = jnp.cos(theta)

    def dsilu(a):
        s = jax.nn.sigmoid(a)
        return s * (1.0 + a * (1.0 - s))

    rbt = k * sth * (rinv * env)
    a12 = lax.dot_general(rbt, wa12_ref[...], (((0,), (0,)), ((), ())),
                          preferred_element_type=F32)
    dw12 = jnp.concatenate([dw1_ref[...], dw2_ref[...]], axis=1)
    ds12 = jnp.dot(dw12, wbdT_ref[...], preferred_element_type=F32)
    da12 = ds12 * dsilu(a12)
    drbt = lax.dot_general(wa12T_ref[...], da12, (((0,), (1,)), ((), ())),
                           preferred_element_type=F32)

    denv = -0.5 * (jnp.pi / CUT) * jnp.sin(jnp.pi * rm / CUT) * mask
    drb_dr = (k * env * ((nvec * (jnp.pi / CUT)) * cth * rinv - sth * rinv * rinv)
              + k * sth * rinv * denv)
    dr = jnp.sum(drbt * drb_dr, axis=0, keepdims=True)
    g = dr * rinv
    zrow = jnp.zeros_like(g)
    dvt_ref[...] = jnp.concatenate([g * vx, g * vy, g * vz, zrow], axis=0)


def _tc_node1(at_ref, emb_ref, wm1_ref, h0_ref, hm1_ref):
    oh = (at_ref[...] == lax.broadcasted_iota(I32, (N, T), 1)).astype(F32)
    h0 = jnp.dot(oh, emb_ref[...], preferred_element_type=F32)
    h0_ref[...] = h0
    hm1_ref[...] = jnp.dot(h0, wm1_ref[...], preferred_element_type=F32)


def _tc_node2(aggp_ref, h0_ref, wu1_ref, wm2_ref, u1_ref, h1_ref, hm2_ref):
    agg = aggp_ref[0] + aggp_ref[1]
    u1 = jnp.dot(agg, wu1_ref[...], preferred_element_type=F32)
    h1 = h0_ref[...] + _silu(u1)
    u1_ref[...] = u1
    h1_ref[...] = h1
    hm2_ref[...] = jnp.dot(h1, wm2_ref[...], preferred_element_type=F32)


def _tc_node3(aggp_ref, h1_ref, wu2_ref, wout_ref, woutT_ref, wu2T_ref,
              e_ref, dg2_ref):
    agg = aggp_ref[0] + aggp_ref[1]
    u2 = jnp.dot(agg, wu2_ref[...], preferred_element_type=F32)
    s = jax.nn.sigmoid(u2)
    h2 = h1_ref[...] + u2 * s
    e_ref[...] = jnp.sum(
        jnp.dot(h2, wout_ref[...], preferred_element_type=F32)).reshape(1, 1)
    du2 = woutT_ref[...] * (s * (1.0 + u2 * (1.0 - s)))
    dg2_ref[...] = jnp.dot(du2, wu2T_ref[...], preferred_element_type=F32)


def _tc_node4(dhmp_ref, u1_ref, wm2T_ref, wu1T_ref, woutT_ref, dg1_ref):
    dhm2 = dhmp_ref[0] + dhmp_ref[1]
    dh1 = woutT_ref[...] + jnp.dot(dhm2, wm2T_ref[...], preferred_element_type=F32)
    u1 = u1_ref[...]
    s = jax.nn.sigmoid(u1)
    du1 = dh1 * (s * (1.0 + u1 * (1.0 - s)))
    dg1_ref[...] = jnp.dot(du1, wu1T_ref[...], preferred_element_type=F32)


def _tc_fsum(fp_ref, out_ref):
    acc = fp_ref[0]
    for k in range(1, NW):
        acc = acc + fp_ref[k]
    out_ref[...] = acc


def _eblk(width):
    return pl.BlockSpec((BLK, width), lambda b: (b, 0))


def _tblk(rows):
    return pl.BlockSpec((rows, BLK), lambda b: (0, b))


def _full(shape):
    return pl.BlockSpec(shape, lambda b: tuple(0 for _ in shape))


def kernel(pos, emb, Wr1a, Wr1b, Wm1, Wu1, Wr2a, Wr2b, Wm2, Wu2, Wout,
           edge_index, atomic_numbers):
    j3 = edge_index[0].reshape(NW, NCH, CH)
    i3 = edge_index[1].reshape(NW, NCH, CH)
    zND = jnp.zeros((N, D), F32)
    at2 = atomic_numbers.reshape(N, 1)
    WoutT = Wout.T
    Wu1T, Wu2T, Wm2T = Wu1.T, Wu2.T, Wm2.T
    Wa12 = jnp.concatenate([Wr1a, Wr2a], axis=1)
    z64 = jnp.zeros((64, D), F32)
    Wbd = jnp.concatenate(
        [jnp.concatenate([Wr1b, z64], axis=0),
         jnp.concatenate([z64, Wr2b], axis=0)], axis=1)
    WbdT = Wbd.T
    Wa12T = Wa12.T

    vec4 = _sc_vec(pos[:, 0], pos[:, 1], pos[:, 2], j3, i3)
    vecT = vec4.reshape(E, 4).T

    w1, w2 = pl.pallas_call(
        _tc_radial_fwd,
        grid=(E // BLK,),
        in_specs=[_tblk(4), _full((NB, D)), _full((D, 2 * D))],
        out_specs=[_eblk(D), _eblk(D)],
        out_shape=[jax.ShapeDtypeStruct((E, D), F32),
                   jax.ShapeDtypeStruct((E, D), F32)],
    )(vecT, Wa12, Wbd)

    h0, hm1 = pl.pallas_call(
        _tc_node1,
        out_shape=[jax.ShapeDtypeStruct((N, D), F32),
                   jax.ShapeDtypeStruct((N, D), F32)],
    )(at2, emb, Wm1)

    agg1p = _sc_fwd_msg(w1.reshape(NW, NCH, CH, D), hm1, j3, i3, zND)

    u1, h1, hm2 = pl.pallas_call(
        _tc_node2,
        out_shape=[jax.ShapeDtypeStruct((N, D), F32)] * 3,
    )(agg1p, h0, Wu1, Wm2)

    agg2p = _sc_fwd_msg(w2.reshape(NW, NCH, CH, D), hm2, j3, i3, zND)

    e11, dg2 = pl.pallas_call(
        _tc_node3,
        out_shape=[jax.ShapeDtypeStruct((1, 1), F32),
                   jax.ShapeDtypeStruct((N, D), F32)],
    )(agg2p, h1, Wu2, Wout, WoutT, Wu2T)

    dw2_4, dhm2p = _sc_bwd2(w2.reshape(NW, NCH, CH, D), hm2, dg2, j3, i3, zND)

    dg1 = pl.pallas_call(
        _tc_node4,
        out_shape=jax.ShapeDtypeStruct((N, D), F32),
    )(dhm2p, u1, Wm2T, Wu1T, WoutT)

    dw1_4 = _sc_bwd1(hm1, dg1, j3, i3)

    dvT = pl.pallas_call(
        _tc_radial_bwd,
        grid=(E // BLK,),
        in_specs=[_tblk(4), _eblk(D), _eblk(D), _full((NB, D)),
                  _full((2 * D, D)), _full((D, NB))],
        out_specs=[_tblk(4)],
        out_shape=[jax.ShapeDtypeStruct((4, E), F32)],
    )(vecT, dw1_4.reshape(E, D), dw2_4.reshape(E, D),
      Wa12, WbdT, Wa12T)[0]
    dv = dvT.T

    fp = _sc_force(dv.reshape(NW, NCH, CH, 4), j3, i3)

    fsum = pl.pallas_call(
        _tc_fsum,
        out_shape=jax.ShapeDtypeStruct((3, N), F32),
    )(fp)

    forces = fsum.T
    energy = e11.reshape(1)
    return (energy, forces)
